# baseline (speedup 1.0000x reference)
import functools

import jax
import jax.numpy as jnp
import numpy as np
from jax import lax
from jax.experimental import pallas as pl
from jax.experimental.pallas import tpu as pltpu
from jax.experimental.pallas import tpu_sc as plsc

EPS = 1e-5
B = 16384
D = 32
LANES = 128
NW = 32
BPW = B // NW
CHUNK = 128
EPR = 4
NCHUNK = BPW // CHUNK
GPC = CHUNK // 16

_MAGIC = np.int32(0x5F3759DF)
_MANT = np.int32(0x007FFFFF)
_ONE_F = np.int32(0x3F800000)
_LN2 = 0.6931471805599453
_SQRT2 = 1.4142135623730951


def _bits(x):
    return lax.bitcast_convert_type(x, jnp.int32)


def _f32(i):
    return lax.bitcast_convert_type(i, jnp.float32)


def _sqrt(x):
    y = _f32(_MAGIC - (_bits(x) >> 1))
    y = y * (1.5 - 0.5 * x * y * y)
    y = y * (1.5 - 0.5 * x * y * y)
    y = y * (1.5 - 0.5 * x * y * y)
    return x * y


def _log(x):
    i = _bits(x)
    e = (i >> 23) - 127
    m = _f32((i & _MANT) | _ONE_F)
    big = m > _SQRT2
    m = jnp.where(big, 0.5 * m, m)
    e = jnp.where(big, e + 1, e)
    t = (m - 1.0) / (m + 1.0)
    t2 = t * t
    p = 2.0 * t * (1.0 + t2 * (1.0 / 3.0 + t2 * (0.2 + t2 * (1.0 / 7.0 + t2 * (1.0 / 9.0)))))
    return e.astype(jnp.float32) * _LN2 + p


def _tanh_pos(x):
    return 1.0 - 2.0 / (jnp.exp(2.0 * x) + 1.0)


def _unit_scale(n):
    return jnp.where(n >= 1.0, (1.0 - EPS) / jnp.maximum(n, 1e-10), jnp.float32(1.0))


def _score_group(s_hh, s_tt, s_rr, s_hw2, s_tr, s_ht, s_hr, b0, b1):
    n_h0 = _sqrt(s_hh)
    sc_h = _unit_scale(n_h0)
    sc_t = _unit_scale(_sqrt(s_tt))
    sc_r = _unit_scale(_sqrt(s_rr))

    n1 = jnp.clip(sc_h * n_h0, 1e-10, 1.0 - 1e-7)
    artanh = 0.5 * _log((1.0 + n1) / (1.0 - n1))
    alpha = artanh / n1 * sc_h

    rt_hw2 = _sqrt(s_hw2)
    nm = jnp.maximum(alpha * rt_hw2, 1e-10)
    mu = alpha * _tanh_pos(nm) / nm
    zeta = mu * _unit_scale(mu * rt_hw2)
    head2 = zeta * zeta * s_hw2

    xy = sc_t * sc_r * s_tr
    x2 = sc_t * sc_t * s_tt
    y2 = sc_r * sc_r * s_rr
    den = jnp.maximum(1.0 + 2.0 * xy + x2 * y2, 1e-10)
    a = (1.0 + 2.0 * xy + y2) * sc_t / den
    b = (1.0 - x2) * sc_r / den
    u2 = a * a * s_tt + 2.0 * a * b * s_tr + b * b * s_rr
    sc_u = _unit_scale(_sqrt(u2))
    a2 = a * sc_u
    b2 = b * sc_u
    tail2 = sc_u * sc_u * u2

    d2 = head2 + tail2 - 2.0 * zeta * (a2 * s_ht + b2 * s_hr)
    axay = (1.0 - head2) * (1.0 - tail2)
    z1 = jnp.maximum(2.0 * d2 / jnp.maximum(axay, 1e-10), 1e-7)
    dist = _log(1.0 + z1 + _sqrt(z1 * (z1 + 2.0)))
    return -dist + b0 + b1


def _body(eh4, rvP, wfP, bias0, bias1, hidx, ridx, tidx, out,
          idx_hc, idx_tc, idx_rc, idx_h4, idx_t4, idx_hv, idx_tv,
          h4, t4, r4, w4, b0_v, b1_v, out_v, sem, semb):
    wid = lax.axis_index("s") * 2 + lax.axis_index("c")
    base = wid * BPW

    pltpu.sync_copy(hidx.at[pl.ds(base, BPW)], idx_hv)
    pltpu.sync_copy(tidx.at[pl.ds(base, BPW)], idx_tv)
    for j in range(NCHUNK):
        pltpu.sync_copy(hidx.at[pl.ds(base + j * CHUNK, CHUNK)], idx_hc.at[j])
        pltpu.sync_copy(tidx.at[pl.ds(base + j * CHUNK, CHUNK)], idx_tc.at[j])
        pltpu.sync_copy(ridx.at[pl.ds(base + j * CHUNK, CHUNK)], idx_rc.at[j])

    for j in range(NCHUNK):
        for k in range(CHUNK // 16):
            sl = pl.ds(j * CHUNK + k * 16, 16)
            idx_h4[j, pl.ds(k * 16, 16)] = idx_hv[sl] >> 2
            idx_t4[j, pl.ds(k * 16, 16)] = idx_tv[sl] >> 2

    bias_copies = []
    for j in range(NCHUNK):
        sl = pl.ds(j * CHUNK, CHUNK)
        bias_copies.append(pltpu.make_async_copy(bias0.at[idx_hc.at[j]], b0_v.at[sl], semb))
        bias_copies.append(pltpu.make_async_copy(bias1.at[idx_tc.at[j]], b1_v.at[sl], semb))
    for c in bias_copies:
        c.start()
    for c in bias_copies:
        c.wait()

    lanes = lax.iota(jnp.int32, 16)

    for cnk in range(NCHUNK):
        copies = [
            pltpu.make_async_copy(eh4.at[idx_h4.at[cnk]], h4, sem),
            pltpu.make_async_copy(eh4.at[idx_t4.at[cnk]], t4, sem),
            pltpu.make_async_copy(rvP.at[idx_rc.at[cnk]], r4, sem),
            pltpu.make_async_copy(wfP.at[idx_rc.at[cnk]], w4, sem),
        ]
        for c in copies:
            c.start()
        for c in copies:
            c.wait()

        def group(g, carry, cnk=cnk):
            row = g * 16 + lanes
            offh = (idx_hv[pl.ds(cnk * CHUNK + g * 16, 16)] & 3) * D
            offt = (idx_tv[pl.ds(cnk * CHUNK + g * 16, 16)] & 3) * D
            z = jnp.zeros((16,), jnp.float32)
            s_hh = z; s_tt = z; s_rr = z; s_hw2 = z; s_tr = z; s_ht = z; s_hr = z
            for dd in range(D):
                dim = jnp.full((16,), dd, jnp.int32)
                hd = plsc.load_gather(h4, [row, offh + dd])
                td = plsc.load_gather(t4, [row, offt + dd])
                rd = plsc.load_gather(r4, [row, dim])
                wd = plsc.load_gather(w4, [row, dim])
                hw = hd * wd
                s_hh += hd * hd
                s_tt += td * td
                s_rr += rd * rd
                s_hw2 += hw * hw
                s_tr += td * rd
                s_ht += hw * td
                s_hr += hw * rd
            b0 = b0_v[pl.ds(cnk * CHUNK + g * 16, 16)]
            b1 = b1_v[pl.ds(cnk * CHUNK + g * 16, 16)]
            score = _score_group(s_hh, s_tt, s_rr, s_hw2, s_tr, s_ht, s_hr, b0, b1)
            out_v[pl.ds(cnk * CHUNK + g * 16, 16)] = score
            return carry

        lax.fori_loop(0, GPC, group, 0)

    pltpu.sync_copy(out_v, out.at[pl.ds(base, BPW)])


_sc_call = functools.partial(
    pl.kernel,
    out_type=jax.ShapeDtypeStruct((B,), jnp.float32),
    mesh=plsc.VectorSubcoreMesh(core_axis_name="c", subcore_axis_name="s"),
    compiler_params=pltpu.CompilerParams(
        needs_layout_passes=False, use_tc_tiling_on_sc=True),
    scratch_types=[
        pltpu.VMEM((NCHUNK, CHUNK), jnp.int32),
        pltpu.VMEM((NCHUNK, CHUNK), jnp.int32),
        pltpu.VMEM((NCHUNK, CHUNK), jnp.int32),
        pltpu.VMEM((NCHUNK, CHUNK), jnp.int32),
        pltpu.VMEM((NCHUNK, CHUNK), jnp.int32),
        pltpu.VMEM((BPW,), jnp.int32),
        pltpu.VMEM((BPW,), jnp.int32),
        pltpu.VMEM((CHUNK, LANES), jnp.float32),
        pltpu.VMEM((CHUNK, LANES), jnp.float32),
        pltpu.VMEM((CHUNK, LANES), jnp.float32),
        pltpu.VMEM((CHUNK, LANES), jnp.float32),
        pltpu.VMEM((BPW,), jnp.float32),
        pltpu.VMEM((BPW,), jnp.float32),
        pltpu.VMEM((BPW,), jnp.float32),
        pltpu.SemaphoreType.DMA,
        pltpu.SemaphoreType.DMA,
    ],
)(_body)


def kernel(Eh, rvh, weight_for_head, bias0, bias1, head_idx, rel_idx, tail_idx):
    pad = ((0, 0), (0, LANES - D))
    NE = Eh.shape[0]
    return _sc_call(
        Eh.reshape(NE // EPR, LANES),
        jnp.pad(rvh, pad), jnp.pad(weight_for_head, pad),
        bias0, bias1,
        head_idx.astype(jnp.int32), rel_idx.astype(jnp.int32),
        tail_idx.astype(jnp.int32),
    )


# speedup vs baseline: 1.3870x; 1.3870x over previous
import functools

import jax
import jax.numpy as jnp
import numpy as np
from jax import lax
from jax.experimental import pallas as pl
from jax.experimental.pallas import tpu as pltpu
from jax.experimental.pallas import tpu_sc as plsc

EPS = 1e-5
B = 16384
D = 32
LANES = 128
NW = 32
BPW = B // NW
CHUNK = 16
NCHUNK = BPW // CHUNK
GPC = CHUNK // 16

_MAGIC = np.int32(0x5F3759DF)
_MANT = np.int32(0x007FFFFF)
_ONE_F = np.int32(0x3F800000)
_LN2 = 0.6931471805599453
_SQRT2 = 1.4142135623730951


def _bits(x):
    return lax.bitcast_convert_type(x, jnp.int32)


def _f32(i):
    return lax.bitcast_convert_type(i, jnp.float32)


def _sqrt(x):
    y = _f32(_MAGIC - (_bits(x) >> 1))
    y = y * (1.5 - 0.5 * x * y * y)
    y = y * (1.5 - 0.5 * x * y * y)
    y = y * (1.5 - 0.5 * x * y * y)
    return x * y


def _log(x):
    i = _bits(x)
    e = (i >> 23) - 127
    m = _f32((i & _MANT) | _ONE_F)
    big = m > _SQRT2
    m = jnp.where(big, 0.5 * m, m)
    e = jnp.where(big, e + 1, e)
    t = (m - 1.0) / (m + 1.0)
    t2 = t * t
    p = 2.0 * t * (1.0 + t2 * (1.0 / 3.0 + t2 * (0.2 + t2 * (1.0 / 7.0 + t2 * (1.0 / 9.0)))))
    return e.astype(jnp.float32) * _LN2 + p


def _tanh_pos(x):
    return 1.0 - 2.0 / (jnp.exp(2.0 * x) + 1.0)


def _unit_scale(n):
    return jnp.where(n >= 1.0, (1.0 - EPS) / jnp.maximum(n, 1e-10), jnp.float32(1.0))


def _score_group(s_hh, s_tt, s_rr, s_hw2, s_tr, s_ht, s_hr, b0, b1):
    n_h0 = _sqrt(s_hh)
    sc_h = _unit_scale(n_h0)
    sc_t = _unit_scale(_sqrt(s_tt))
    sc_r = _unit_scale(_sqrt(s_rr))

    n1 = jnp.clip(sc_h * n_h0, 1e-10, 1.0 - 1e-7)
    artanh = 0.5 * _log((1.0 + n1) / (1.0 - n1))
    alpha = artanh / n1 * sc_h

    rt_hw2 = _sqrt(s_hw2)
    nm = jnp.maximum(alpha * rt_hw2, 1e-10)
    mu = alpha * _tanh_pos(nm) / nm
    zeta = mu * _unit_scale(mu * rt_hw2)
    head2 = zeta * zeta * s_hw2

    xy = sc_t * sc_r * s_tr
    x2 = sc_t * sc_t * s_tt
    y2 = sc_r * sc_r * s_rr
    den = jnp.maximum(1.0 + 2.0 * xy + x2 * y2, 1e-10)
    a = (1.0 + 2.0 * xy + y2) * sc_t / den
    b = (1.0 - x2) * sc_r / den
    u2 = a * a * s_tt + 2.0 * a * b * s_tr + b * b * s_rr
    sc_u = _unit_scale(_sqrt(u2))
    a2 = a * sc_u
    b2 = b * sc_u
    tail2 = sc_u * sc_u * u2

    d2 = head2 + tail2 - 2.0 * zeta * (a2 * s_ht + b2 * s_hr)
    axay = (1.0 - head2) * (1.0 - tail2)
    z1 = jnp.maximum(2.0 * d2 / jnp.maximum(axay, 1e-10), 1e-7)
    dist = _log(1.0 + z1 + _sqrt(z1 * (z1 + 2.0)))
    return -dist + b0 + b1


def _body(eh, rvP, wfP, bias0, bias1, hidx, ridx, tidx, out,
          idx_hc, idx_tc, idx_rc, idx_hv, idx_tv,
          h8a, t8a, r4a, w4a, h8b, t8b, r4b, w4b,
          b0_v, b1_v, out_v, semA, semB, semb):
    wid = lax.axis_index("s") * 2 + lax.axis_index("c")
    base = wid * BPW

    pltpu.sync_copy(hidx.at[pl.ds(base, BPW)], idx_hv)
    pltpu.sync_copy(tidx.at[pl.ds(base, BPW)], idx_tv)
    for j in range(NCHUNK):
        pltpu.sync_copy(hidx.at[pl.ds(base + j * CHUNK, CHUNK)], idx_hc.at[j])
        pltpu.sync_copy(tidx.at[pl.ds(base + j * CHUNK, CHUNK)], idx_tc.at[j])
        pltpu.sync_copy(ridx.at[pl.ds(base + j * CHUNK, CHUNK)], idx_rc.at[j])

    bias_copies = []
    for j in range(NCHUNK):
        sl = pl.ds(j * CHUNK, CHUNK)
        bias_copies.append(pltpu.make_async_copy(bias0.at[idx_hc.at[j]], b0_v.at[sl], semb))
        bias_copies.append(pltpu.make_async_copy(bias1.at[idx_tc.at[j]], b1_v.at[sl], semb))
    for c in bias_copies:
        c.start()
    for c in bias_copies:
        c.wait()

    lanes = lax.iota(jnp.int32, 16)

    def issue(cnk, h8, t8, r4, w4, s):
        pltpu.make_async_copy(rvP.at[idx_rc.at[cnk]], r4, s).start()
        pltpu.make_async_copy(wfP.at[idx_rc.at[cnk]], w4, s).start()
        ih = idx_hv[pl.ds(cnk * CHUNK, 16)]
        it = idx_tv[pl.ds(cnk * CHUNK, 16)]
        for k in range(16):
            rh = jnp.max(jnp.where(lanes == k, ih, 0))
            rt = jnp.max(jnp.where(lanes == k, it, 0))
            oh = pl.multiple_of((rh >> 3) * 8, 8)
            ot = pl.multiple_of((rt >> 3) * 8, 8)
            pltpu.make_async_copy(eh.at[pl.ds(oh, 8), :], h8.at[pl.ds(k * 8, 8)], s).start()
            pltpu.make_async_copy(eh.at[pl.ds(ot, 8), :], t8.at[pl.ds(k * 8, 8)], s).start()

    def wait_all(cnk, h8, t8, r4, w4, s):
        pltpu.make_async_copy(eh.at[pl.ds(0, CHUNK * 8), :], h8, s).wait()
        pltpu.make_async_copy(eh.at[pl.ds(0, CHUNK * 8), :], t8, s).wait()
        pltpu.make_async_copy(rvP.at[idx_rc.at[cnk]], r4, s).wait()
        pltpu.make_async_copy(wfP.at[idx_rc.at[cnk]], w4, s).wait()

    def compute(cnk, h8, t8, r4, w4):
        ih = idx_hv[pl.ds(cnk * CHUNK, 16)]
        it = idx_tv[pl.ds(cnk * CHUNK, 16)]
        rowh = lanes * 8 + (ih & 7)
        rowt = lanes * 8 + (it & 7)
        z = jnp.zeros((16,), jnp.float32)
        s_hh = z; s_tt = z; s_rr = z; s_hw2 = z; s_tr = z; s_ht = z; s_hr = z
        for dd in range(D):
            dim = jnp.full((16,), dd, jnp.int32)
            hd = plsc.load_gather(h8, [rowh, dim])
            td = plsc.load_gather(t8, [rowt, dim])
            rd = plsc.load_gather(r4, [lanes, dim])
            wd = plsc.load_gather(w4, [lanes, dim])
            hw = hd * wd
            s_hh += hd * hd
            s_tt += td * td
            s_rr += rd * rd
            s_hw2 += hw * hw
            s_tr += td * rd
            s_ht += hw * td
            s_hr += hw * rd
        b0 = b0_v[pl.ds(cnk * CHUNK, 16)]
        b1 = b1_v[pl.ds(cnk * CHUNK, 16)]
        score = _score_group(s_hh, s_tt, s_rr, s_hw2, s_tr, s_ht, s_hr, b0, b1)
        out_v[pl.ds(cnk * CHUNK, 16)] = score

    issue(0, h8a, t8a, r4a, w4a, semA)

    def pipe(i, carry):
        c0 = i * 2
        c1 = c0 + 1
        issue(c1, h8b, t8b, r4b, w4b, semB)
        wait_all(c0, h8a, t8a, r4a, w4a, semA)
        compute(c0, h8a, t8a, r4a, w4a)

        @pl.when(c1 + 1 < NCHUNK)
        def _():
            issue(c1 + 1, h8a, t8a, r4a, w4a, semA)

        wait_all(c1, h8b, t8b, r4b, w4b, semB)
        compute(c1, h8b, t8b, r4b, w4b)
        return carry

    lax.fori_loop(0, NCHUNK // 2, pipe, 0)

    pltpu.sync_copy(out_v, out.at[pl.ds(base, BPW)])


_sc_call = functools.partial(
    pl.kernel,
    out_type=jax.ShapeDtypeStruct((B,), jnp.float32),
    mesh=plsc.VectorSubcoreMesh(core_axis_name="c", subcore_axis_name="s"),
    compiler_params=pltpu.CompilerParams(
        needs_layout_passes=False, use_tc_tiling_on_sc=True),
    scratch_types=[
        pltpu.VMEM((NCHUNK, CHUNK), jnp.int32),
        pltpu.VMEM((NCHUNK, CHUNK), jnp.int32),
        pltpu.VMEM((NCHUNK, CHUNK), jnp.int32),
        pltpu.VMEM((BPW,), jnp.int32),
        pltpu.VMEM((BPW,), jnp.int32),
        pltpu.VMEM((CHUNK * 8, D), jnp.float32),
        pltpu.VMEM((CHUNK * 8, D), jnp.float32),
        pltpu.VMEM((CHUNK, LANES), jnp.float32),
        pltpu.VMEM((CHUNK, LANES), jnp.float32),
        pltpu.VMEM((CHUNK * 8, D), jnp.float32),
        pltpu.VMEM((CHUNK * 8, D), jnp.float32),
        pltpu.VMEM((CHUNK, LANES), jnp.float32),
        pltpu.VMEM((CHUNK, LANES), jnp.float32),
        pltpu.VMEM((BPW,), jnp.float32),
        pltpu.VMEM((BPW,), jnp.float32),
        pltpu.VMEM((BPW,), jnp.float32),
        pltpu.SemaphoreType.DMA,
        pltpu.SemaphoreType.DMA,
        pltpu.SemaphoreType.DMA,
    ],
)(_body)


def kernel(Eh, rvh, weight_for_head, bias0, bias1, head_idx, rel_idx, tail_idx):
    pad = ((0, 0), (0, LANES - D))
    return _sc_call(
        Eh, jnp.pad(rvh, pad), jnp.pad(weight_for_head, pad),
        bias0, bias1,
        head_idx.astype(jnp.int32), rel_idx.astype(jnp.int32),
        tail_idx.astype(jnp.int32),
    )
